# baseline (speedup 1.0000x reference)
import functools

import jax
import jax.numpy as jnp
from jax import lax
from jax.experimental import pallas as pl
from jax.experimental.pallas import tpu as pltpu
from jax.experimental.pallas import tpu_sc as plsc

_B, _S, _D = 4, 2048, 1024
_NW = 32
_P = _S // _NW
_CH = 16
_NCH = _P // _CH
_NBUF = 3
_OBUF = 2
_LANES = 16
_SLICES = _D // _LANES

_mesh = plsc.VectorSubcoreMesh(core_axis_name="c", subcore_axis_name="s")


@functools.partial(
    pl.kernel,
    mesh=_mesh,
    out_type=jax.ShapeDtypeStruct((_B, _S, _D), jnp.float32),
    scratch_types=[
        pltpu.VMEM((_NBUF, _CH, _D), jnp.float32),
        pltpu.VMEM((_OBUF, _CH, _D), jnp.float32),
        pltpu.VMEM((2, _CH, _D), jnp.float32),
        pltpu.SemaphoreType.DMA((_NBUF,)),
        pltpu.SemaphoreType.DMA((2,)),
        pltpu.SemaphoreType.DMA((_OBUF,)),
    ],
)
def _sc_add(x_hbm, tbl_hbm, out_hbm, xr, orr, tr, si, st, so):
    cid = lax.axis_index("c")
    sid = lax.axis_index("s")
    wid = sid * 2 + cid
    base = wid * _P

    items = [(c, b) for c in range(_NCH) for b in range(_B)]
    n = len(items)

    def x_src(item):
        c, b = item
        return x_hbm.at[b, pl.ds(base + c * _CH, _CH)]

    def out_dst(item):
        c, b = item
        return out_hbm.at[b, pl.ds(base + c * _CH, _CH)]

    x_in = [None] * n
    wb = [None] * n

    pltpu.async_copy(tbl_hbm.at[pl.ds(base, _CH)], tr.at[0], st.at[0])
    for i in range(_NBUF - 1):
        x_in[i] = pltpu.async_copy(x_src(items[i]), xr.at[i], si.at[i])

    for i, (c, b) in enumerate(items):
        buf = xr.at[i % _NBUF]
        obuf = orr.at[i % _OBUF]
        tbuf = tr.at[c % 2]
        j = i + _NBUF - 1
        if j < n:
            x_in[j] = pltpu.async_copy(
                x_src(items[j]), xr.at[j % _NBUF], si.at[j % _NBUF])
        if b == _B - 1 and c + 1 < _NCH:
            pltpu.async_copy(
                tbl_hbm.at[pl.ds(base + (c + 1) * _CH, _CH)],
                tr.at[(c + 1) % 2], st.at[(c + 1) % 2])
        x_in[i].wait()
        if b == 0:
            pltpu.make_async_copy(
                tbl_hbm.at[pl.ds(base + c * _CH, _CH)], tbuf,
                st.at[c % 2]).wait()
        if wb[i - _OBUF] is not None:
            wb[i - _OBUF].wait()

        def _rows(r2, _):
            for rr in range(2):
                for k in range(_SLICES):
                    sl = pl.ds(k * _LANES, _LANES)
                    obuf[r2 * 2 + rr, sl] = (
                        buf[r2 * 2 + rr, sl] + tbuf[r2 * 2 + rr, sl])
            return 0

        lax.fori_loop(0, _CH // 2, _rows, 0)
        wb[i] = pltpu.async_copy(obuf, out_dst(items[i]), so.at[i % _OBUF])

    for i in range(n - _OBUF, n):
        wb[i].wait()


def kernel(x, pos_table, maxlen):
    return _sc_add(x, pos_table)
